# baseline (speedup 1.0000x reference)
import functools

import jax
import jax.numpy as jnp
from jax import lax
from jax.experimental import pallas as pl
from jax.experimental.pallas import tpu as pltpu

TE = 1000
TN = 1000


def _fixed(shape):
    return pl.BlockSpec(shape, lambda i: tuple(0 for _ in shape))


def _stats_body(h_ref, o_ref):
    @pl.when(pl.program_id(0) == 0)
    def _():
        o_ref[...] = jnp.zeros_like(o_ref)

    hb = h_ref[...]
    o_ref[0:1, :] += jnp.sum(hb, axis=0, keepdims=True)
    o_ref[1:2, :] += jnp.sum(hb * hb, axis=0, keepdims=True)


def _stats(h):
    n, d = h.shape
    return pl.pallas_call(
        _stats_body,
        grid=(n // TN,),
        in_specs=[pl.BlockSpec((TN, d), lambda i: (i, 0))],
        out_specs=_fixed((2, d)),
        out_shape=jax.ShapeDtypeStruct((2, d), jnp.float32),
    )(h)


def _bn_affine(stats, gamma, beta, n):
    mean = stats[0] / n
    var = stats[1] / n - mean * mean
    s = gamma * jax.lax.rsqrt(var + 1e-5)
    shift = beta - mean * s
    return s[None, :], shift[None, :]


def _mlp_enc_body(relu_out, s_ref, sh_ref, b1_ref, b2_ref, xi_ref, ea_ref,
                  w1x_ref, w1e_ref, w2_ref, m_ref):
    xb = xi_ref[...] * s_ref[...] + sh_ref[...]
    h1 = jnp.dot(xb, w1x_ref[...], preferred_element_type=jnp.float32)
    h1 += jnp.dot(ea_ref[...], w1e_ref[...], preferred_element_type=jnp.float32)
    h1 = jnp.maximum(h1 + b1_ref[...], 0.0)
    m = jnp.dot(h1, w2_ref[...], preferred_element_type=jnp.float32) + b2_ref[...]
    if relu_out:
        m = jnp.maximum(m, 0.0)
    m_ref[...] = m


def _mlp_enc(xi, ea, s, sh, w1x, w1e, b1, w2, b2, relu_out):
    e, d = xi.shape
    de = ea.shape[1]
    hid = w1x.shape[1]
    hout = w2.shape[1]
    return pl.pallas_call(
        functools.partial(_mlp_enc_body, relu_out),
        grid=(e // TE,),
        in_specs=[_fixed((1, d)), _fixed((1, d)), _fixed((1, hid)),
                  _fixed((1, hout)),
                  pl.BlockSpec((TE, d), lambda i: (i, 0)),
                  pl.BlockSpec((TE, de), lambda i: (i, 0)),
                  _fixed((d, hid)), _fixed((de, hid)), _fixed((hid, hout))],
        out_specs=pl.BlockSpec((TE, hout), lambda i: (i, 0)),
        out_shape=jax.ShapeDtypeStruct((e, hout), jnp.float32),
    )(s, sh, b1, b2, xi, ea, w1x, w1e, w2)


def _mlp_dec_body(relu_out, s_ref, sh_ref, b1_ref, b2_ref, xi_ref, xj_ref,
                  w1a_ref, w1b_ref, w2_ref, m_ref):
    bni = xi_ref[...] * s_ref[...] + sh_ref[...]
    bnj = xj_ref[...] * s_ref[...] + sh_ref[...]
    h1 = jnp.dot(bni, w1a_ref[...], preferred_element_type=jnp.float32)
    h1 += jnp.dot(bnj - bni, w1b_ref[...], preferred_element_type=jnp.float32)
    h1 = jnp.maximum(h1 + b1_ref[...], 0.0)
    m = jnp.dot(h1, w2_ref[...], preferred_element_type=jnp.float32) + b2_ref[...]
    if relu_out:
        m = jnp.maximum(m, 0.0)
    m_ref[...] = m


def _mlp_dec(xi, xj, s, sh, w1a, w1b, b1, w2, b2, relu_out):
    e, d = xi.shape
    hid = w1a.shape[1]
    hout = w2.shape[1]
    return pl.pallas_call(
        functools.partial(_mlp_dec_body, relu_out),
        grid=(e // TE,),
        in_specs=[_fixed((1, d)), _fixed((1, d)), _fixed((1, hid)),
                  _fixed((1, hout)),
                  pl.BlockSpec((TE, d), lambda i: (i, 0)),
                  pl.BlockSpec((TE, d), lambda i: (i, 0)),
                  _fixed((d, hid)), _fixed((d, hid)), _fixed((hid, hout))],
        out_specs=pl.BlockSpec((TE, hout), lambda i: (i, 0)),
        out_shape=jax.ShapeDtypeStruct((e, hout), jnp.float32),
    )(s, sh, b1, b2, xi, xj, w1a, w1b, w2)


def _combine_body(has_res, s_ref, sh_ref, br_ref, sums_ref, cnt_ref, h_ref,
                  wr_ref, o_ref, st_ref):
    cnt = jnp.maximum(cnt_ref[...][:, 0:1], 1.0)
    agg = sums_ref[...] / cnt
    hb = h_ref[...] * s_ref[...] + sh_ref[...]
    if has_res:
        res = jnp.dot(hb, wr_ref[...], preferred_element_type=jnp.float32)
        res = res + br_ref[...]
    else:
        res = hb
    hn = agg + res
    o_ref[...] = hn

    @pl.when(pl.program_id(0) == 0)
    def _():
        st_ref[...] = jnp.zeros_like(st_ref)

    st_ref[0:1, :] += jnp.sum(hn, axis=0, keepdims=True)
    st_ref[1:2, :] += jnp.sum(hn * hn, axis=0, keepdims=True)


def _combine(sums, cnt16, h, s, sh, wr, br, hout):
    n, d = h.shape
    if wr is None:
        wr = jnp.zeros((d, hout), jnp.float32)
        br = jnp.zeros((1, hout), jnp.float32)
        has_res = False
    else:
        has_res = True
    return pl.pallas_call(
        functools.partial(_combine_body, has_res),
        grid=(n // TN,),
        in_specs=[_fixed((1, d)), _fixed((1, d)), _fixed((1, hout)),
                  pl.BlockSpec((TN, hout), lambda i: (i, 0)),
                  pl.BlockSpec((TN, 16), lambda i: (i, 0)),
                  pl.BlockSpec((TN, d), lambda i: (i, 0)),
                  _fixed((d, hout))],
        out_specs=[pl.BlockSpec((TN, hout), lambda i: (i, 0)),
                   _fixed((2, hout))],
        out_shape=[jax.ShapeDtypeStruct((n, hout), jnp.float32),
                   jax.ShapeDtypeStruct((2, hout), jnp.float32)],
    )(s, sh, br, sums, cnt16, h, wr)


def _znodes_body(h_ref, eps_ref, wmu_ref, bmu_ref, wlv_ref, blv_ref,
                 mu_ref, lv_ref, z_ref, st_ref):
    hb = h_ref[...]
    mu = jnp.dot(hb, wmu_ref[...], preferred_element_type=jnp.float32) + bmu_ref[...]
    lv = jnp.dot(hb, wlv_ref[...], preferred_element_type=jnp.float32) + blv_ref[...]
    z = mu + eps_ref[...] * jnp.exp(0.5 * lv)
    mu_ref[...] = mu
    lv_ref[...] = lv
    z_ref[...] = z

    @pl.when(pl.program_id(0) == 0)
    def _():
        st_ref[...] = jnp.zeros_like(st_ref)

    st_ref[0:1, :] += jnp.sum(z, axis=0, keepdims=True)
    st_ref[1:2, :] += jnp.sum(z * z, axis=0, keepdims=True)


def _znodes(h, eps, wmu, bmu, wlv, blv):
    n, d = h.shape
    lat = wmu.shape[1]
    row = pl.BlockSpec((TN, lat), lambda i: (i, 0))
    return pl.pallas_call(
        _znodes_body,
        grid=(n // TN,),
        in_specs=[pl.BlockSpec((TN, d), lambda i: (i, 0)), row,
                  _fixed((d, lat)), _fixed((1, lat)),
                  _fixed((d, lat)), _fixed((1, lat))],
        out_specs=[row, row, row, _fixed((2, lat))],
        out_shape=[jax.ShapeDtypeStruct((n, lat), jnp.float32)] * 3
        + [jax.ShapeDtypeStruct((2, lat), jnp.float32)],
    )(h, eps, wmu, bmu, wlv, blv)


def _ep_body(za_ref, zb_ref, w1a_ref, w1b_ref, b1_ref, w2_ref, b2_ref,
             w3_ref, b3_ref, wda_ref, wdb_ref, bd_ref, o_ref):
    a = za_ref[...]
    b = zb_ref[...]
    mn = jnp.minimum(a, b)
    mx = jnp.maximum(a, b)
    h1 = jnp.dot(mn, w1a_ref[...], preferred_element_type=jnp.float32)
    h1 += jnp.dot(mx, w1b_ref[...], preferred_element_type=jnp.float32)
    h1 = jnp.maximum(h1 + b1_ref[...], 0.0)
    h2 = jnp.dot(h1, w2_ref[...], preferred_element_type=jnp.float32)
    h2 = jnp.maximum(h2 + b2_ref[...] + h1, 0.0)
    o = jnp.dot(h2, w3_ref[...], preferred_element_type=jnp.float32) + b3_ref[...]
    o += jnp.dot(mn, wda_ref[...], preferred_element_type=jnp.float32)
    o += jnp.dot(mx, wdb_ref[...], preferred_element_type=jnp.float32)
    o_ref[...] = o + bd_ref[...]


def _mlp_ep(za, zb, w1a, w1b, b1, w2, b2, w3, b3, wda, wdb, bd):
    e, lat = za.shape
    hid = w2.shape[0]
    de = w3.shape[1]
    row = pl.BlockSpec((TE, lat), lambda i: (i, 0))
    return pl.pallas_call(
        _ep_body,
        grid=(e // TE,),
        in_specs=[row, row,
                  _fixed((lat, hid)), _fixed((lat, hid)), _fixed((1, hid)),
                  _fixed((hid, hid)), _fixed((1, hid)),
                  _fixed((hid, de)), _fixed((1, de)),
                  _fixed((lat, de)), _fixed((lat, de)), _fixed((1, de))],
        out_specs=pl.BlockSpec((TE, de), lambda i: (i, 0)),
        out_shape=jax.ShapeDtypeStruct((e, de), jnp.float32),
    )(za, zb, w1a, w1b, b1, w2, b2, w3, b3, wda, wdb, bd)


def _gather_rows(table, idx):
    return jnp.take(table, idx, axis=0)


def _scatter_sum(m, dst, n):
    return jax.ops.segment_sum(m, dst, num_segments=n)


def _edge_counts16(dst, n):
    cnt = jax.ops.segment_sum(jnp.ones((dst.shape[0],), jnp.float32), dst,
                              num_segments=n)
    return jnp.tile(cnt[:, None], (1, 16))


def _t(lp):
    return jnp.asarray(lp[0]).T, jnp.asarray(lp[1])[None, :]


def kernel(x, edge_index, edge_attr, params):
    n, din = x.shape
    e = edge_index.shape[1]
    src = edge_index[0]
    dst = edge_index[1]
    p = params

    cnt16 = _edge_counts16(dst, n)

    h = x
    for i in range(3):
        blk = p['eblk%d' % i]
        gamma, beta = p['ebn'][i]
        d = h.shape[1]
        st = _stats(h)
        s, sh = _bn_affine(st, gamma, beta, float(n))
        w1, b1 = _t(blk['nn1'])
        w1x, w1e = w1[:d], w1[d:]
        w2, b2 = _t(blk['nn2'])
        xi = _gather_rows(h, dst)
        m = _mlp_enc(xi, edge_attr, s, sh, w1x, w1e, b1, w2, b2, True)
        sums = _scatter_sum(m, dst, n)
        if 'res' in blk:
            wr, br = _t(blk['res'])
        else:
            wr, br = None, None
        h, _ = _combine(sums, cnt16, h, s, sh, wr, br, w2.shape[1])

    wmu, bmu = _t(p['fc_mu'])
    wlv, blv = _t(p['fc_logvar'])
    eps = jax.random.normal(jax.random.key(42), (n, wmu.shape[1]),
                            dtype=jnp.float32)
    mu, logvar, z, zst = _znodes(h, eps, wmu, bmu, wlv, blv)

    h = z
    prev_st = zst
    zi = None
    zj = None
    for i in range(2):
        blk = p['dblk%d' % i]
        gamma, beta = p['dbn'][i]
        d = h.shape[1]
        final = i == 1
        s, sh = _bn_affine(prev_st, gamma, beta, float(n))
        w1, b1 = _t(blk['nn1'])
        w1a, w1b = w1[:d], w1[d:]
        w2, b2 = _t(blk['nn2'])
        xi = _gather_rows(h, dst)
        xj = _gather_rows(h, src)
        if i == 0:
            zi, zj = xi, xj
        m = _mlp_dec(xi, xj, s, sh, w1a, w1b, b1, w2, b2, not final)
        sums = _scatter_sum(m, dst, n)
        wr, br = _t(blk['res'])
        h, prev_st = _combine(sums, cnt16, h, s, sh, wr, br, w2.shape[1])
    x_recon = h

    ep = p['ep']
    w1, b1 = _t(ep['fc1'])
    lat = z.shape[1]
    w1a, w1b = w1[:lat], w1[lat:]
    w2, b2 = _t(ep['fc2'])
    w3, b3 = _t(ep['fc3'])
    wd, bd = _t(ep['fc_direct'])
    wda, wdb = wd[:lat], wd[lat:]
    pred_edge = _mlp_ep(zj, zi, w1a, w1b, b1, w2, b2, w3, b3, wda, wdb, bd)

    return x_recon, pred_edge, mu, logvar


# speedup vs baseline: 2.6841x; 2.6841x over previous
import functools

import jax
import jax.numpy as jnp
from jax import lax
from jax.experimental import pallas as pl
from jax.experimental.pallas import tpu as pltpu
from jax.experimental.pallas import tpu_sc as plsc

TE = 1000
TN = 1000
GW = 128
SCH = 80


def _fixed(shape):
    return pl.BlockSpec(shape, lambda i: tuple(0 for _ in shape))


def _stats_body(h_ref, o_ref):
    @pl.when(pl.program_id(0) == 0)
    def _():
        o_ref[...] = jnp.zeros_like(o_ref)

    hb = h_ref[...]
    o_ref[0:1, :] += jnp.sum(hb, axis=0, keepdims=True)
    o_ref[1:2, :] += jnp.sum(hb * hb, axis=0, keepdims=True)


def _stats(h):
    n, d = h.shape
    return pl.pallas_call(
        _stats_body,
        grid=(n // TN,),
        in_specs=[pl.BlockSpec((TN, d), lambda i: (i, 0))],
        out_specs=_fixed((2, d)),
        out_shape=jax.ShapeDtypeStruct((2, d), jnp.float32),
    )(h)


def _bn_affine(stats, gamma, beta, n):
    mean = stats[0] / n
    var = stats[1] / n - mean * mean
    s = gamma * jax.lax.rsqrt(var + 1e-5)
    shift = beta - mean * s
    return s[None, :], shift[None, :]


def _mlp_enc_body(relu_out, s_ref, sh_ref, b1_ref, b2_ref, xi_ref, ea_ref,
                  w1x_ref, w1e_ref, w2_ref, m_ref):
    xb = xi_ref[...] * s_ref[...] + sh_ref[...]
    h1 = jnp.dot(xb, w1x_ref[...], preferred_element_type=jnp.float32)
    h1 += jnp.dot(ea_ref[...], w1e_ref[...], preferred_element_type=jnp.float32)
    h1 = jnp.maximum(h1 + b1_ref[...], 0.0)
    m = jnp.dot(h1, w2_ref[...], preferred_element_type=jnp.float32) + b2_ref[...]
    if relu_out:
        m = jnp.maximum(m, 0.0)
    hh = m.shape[1] // 2
    m_ref[0, :, :] = m[:, :hh]
    m_ref[1, :, :] = m[:, hh:]


def _mlp_enc(xi, ea, s, sh, w1x, w1e, b1, w2, b2, relu_out):
    e, d = xi.shape
    de = ea.shape[1]
    hid = w1x.shape[1]
    hout = w2.shape[1]
    hh = hout // 2
    return pl.pallas_call(
        functools.partial(_mlp_enc_body, relu_out),
        grid=(e // TE,),
        in_specs=[_fixed((1, d)), _fixed((1, d)), _fixed((1, hid)),
                  _fixed((1, hout)),
                  pl.BlockSpec((TE, d), lambda i: (i, 0)),
                  pl.BlockSpec((TE, de), lambda i: (i, 0)),
                  _fixed((d, hid)), _fixed((de, hid)), _fixed((hid, hout))],
        out_specs=pl.BlockSpec((2, TE, hh), lambda i: (0, i, 0)),
        out_shape=jax.ShapeDtypeStruct((2, e, hh), jnp.float32),
    )(s, sh, b1, b2, xi, ea, w1x, w1e, w2)


def _mlp_dec_body(relu_out, deff, s_ref, sh_ref, b1_ref, b2_ref, xi_ref,
                  xj_ref, w1a_ref, w1b_ref, w2_ref, m_ref):
    bni = xi_ref[...][:, :deff] * s_ref[...] + sh_ref[...]
    bnj = xj_ref[...][:, :deff] * s_ref[...] + sh_ref[...]
    h1 = jnp.dot(bni, w1a_ref[...], preferred_element_type=jnp.float32)
    h1 += jnp.dot(bnj - bni, w1b_ref[...], preferred_element_type=jnp.float32)
    h1 = jnp.maximum(h1 + b1_ref[...], 0.0)
    m = jnp.dot(h1, w2_ref[...], preferred_element_type=jnp.float32) + b2_ref[...]
    if relu_out:
        m = jnp.maximum(m, 0.0)
    hh = m.shape[1] // 2
    m_ref[0, :, :] = m[:, :hh]
    m_ref[1, :, :] = m[:, hh:]


def _mlp_dec(xi, xj, s, sh, w1a, w1b, b1, w2, b2, relu_out):
    e, dt = xi.shape
    d = w1a.shape[0]
    hid = w1a.shape[1]
    hout = w2.shape[1]
    hh = hout // 2
    return pl.pallas_call(
        functools.partial(_mlp_dec_body, relu_out, d),
        grid=(e // TE,),
        in_specs=[_fixed((1, d)), _fixed((1, d)), _fixed((1, hid)),
                  _fixed((1, hout)),
                  pl.BlockSpec((TE, dt), lambda i: (i, 0)),
                  pl.BlockSpec((TE, dt), lambda i: (i, 0)),
                  _fixed((d, hid)), _fixed((d, hid)), _fixed((hid, hout))],
        out_specs=pl.BlockSpec((2, TE, hh), lambda i: (0, i, 0)),
        out_shape=jax.ShapeDtypeStruct((2, e, hh), jnp.float32),
    )(s, sh, b1, b2, xi, xj, w1a, w1b, w2)


def _combine_body(has_res, s_ref, sh_ref, br_ref, sums_ref, cnt_ref, h_ref,
                  wr_ref, o_ref, st_ref):
    cnt = jnp.maximum(cnt_ref[...][:, 0:1], 1.0)
    agg = jnp.concatenate([sums_ref[0], sums_ref[1]], axis=-1) / cnt
    hb = h_ref[...] * s_ref[...] + sh_ref[...]
    if has_res:
        res = jnp.dot(hb, wr_ref[...], preferred_element_type=jnp.float32)
        res = res + br_ref[...]
    else:
        res = hb
    hn = agg + res
    o_ref[...] = hn

    @pl.when(pl.program_id(0) == 0)
    def _():
        st_ref[...] = jnp.zeros_like(st_ref)

    st_ref[0:1, :] += jnp.sum(hn, axis=0, keepdims=True)
    st_ref[1:2, :] += jnp.sum(hn * hn, axis=0, keepdims=True)


def _combine(sums, cnt16, h, s, sh, wr, br, hout):
    n, d = h.shape
    if wr is None:
        wr = jnp.zeros((d, hout), jnp.float32)
        br = jnp.zeros((1, hout), jnp.float32)
        has_res = False
    else:
        has_res = True
    return pl.pallas_call(
        functools.partial(_combine_body, has_res),
        grid=(n // TN,),
        in_specs=[_fixed((1, d)), _fixed((1, d)), _fixed((1, hout)),
                  pl.BlockSpec((2, TN, hout // 2), lambda i: (0, i, 0)),
                  pl.BlockSpec((TN, 16), lambda i: (i, 0)),
                  pl.BlockSpec((TN, d), lambda i: (i, 0)),
                  _fixed((d, hout))],
        out_specs=[pl.BlockSpec((TN, hout), lambda i: (i, 0)),
                   _fixed((2, hout))],
        out_shape=[jax.ShapeDtypeStruct((n, hout), jnp.float32),
                   jax.ShapeDtypeStruct((2, hout), jnp.float32)],
    )(s, sh, br, sums, cnt16, h, wr)


def _znodes_body(h_ref, eps_ref, wmu_ref, bmu_ref, wlv_ref, blv_ref,
                 mu_ref, lv_ref, z_ref, zp_ref, st_ref):
    hb = h_ref[...]
    mu = jnp.dot(hb, wmu_ref[...], preferred_element_type=jnp.float32) + bmu_ref[...]
    lv = jnp.dot(hb, wlv_ref[...], preferred_element_type=jnp.float32) + blv_ref[...]
    z = mu + eps_ref[...] * jnp.exp(0.5 * lv)
    mu_ref[...] = mu
    lv_ref[...] = lv
    z_ref[...] = z
    zp_ref[...] = jnp.concatenate([z, jnp.zeros_like(z)], axis=-1)

    @pl.when(pl.program_id(0) == 0)
    def _():
        st_ref[...] = jnp.zeros_like(st_ref)

    st_ref[0:1, :] += jnp.sum(z, axis=0, keepdims=True)
    st_ref[1:2, :] += jnp.sum(z * z, axis=0, keepdims=True)


def _znodes(h, eps, wmu, bmu, wlv, blv):
    n, d = h.shape
    lat = wmu.shape[1]
    row = pl.BlockSpec((TN, lat), lambda i: (i, 0))
    return pl.pallas_call(
        _znodes_body,
        grid=(n // TN,),
        in_specs=[pl.BlockSpec((TN, d), lambda i: (i, 0)), row,
                  _fixed((d, lat)), _fixed((1, lat)),
                  _fixed((d, lat)), _fixed((1, lat))],
        out_specs=[row, row, row,
                   pl.BlockSpec((TN, 2 * lat), lambda i: (i, 0)),
                   _fixed((2, lat))],
        out_shape=[jax.ShapeDtypeStruct((n, lat), jnp.float32)] * 3
        + [jax.ShapeDtypeStruct((n, 2 * lat), jnp.float32),
           jax.ShapeDtypeStruct((2, lat), jnp.float32)],
    )(h, eps, wmu, bmu, wlv, blv)


def _ep_body(lat, za_ref, zb_ref, w1a_ref, w1b_ref, b1_ref, w2_ref, b2_ref,
             w3_ref, b3_ref, wda_ref, wdb_ref, bd_ref, o_ref):
    a = za_ref[...][:, :lat]
    b = zb_ref[...][:, :lat]
    mn = jnp.minimum(a, b)
    mx = jnp.maximum(a, b)
    h1 = jnp.dot(mn, w1a_ref[...], preferred_element_type=jnp.float32)
    h1 += jnp.dot(mx, w1b_ref[...], preferred_element_type=jnp.float32)
    h1 = jnp.maximum(h1 + b1_ref[...], 0.0)
    h2 = jnp.dot(h1, w2_ref[...], preferred_element_type=jnp.float32)
    h2 = jnp.maximum(h2 + b2_ref[...] + h1, 0.0)
    o = jnp.dot(h2, w3_ref[...], preferred_element_type=jnp.float32) + b3_ref[...]
    o += jnp.dot(mn, wda_ref[...], preferred_element_type=jnp.float32)
    o += jnp.dot(mx, wdb_ref[...], preferred_element_type=jnp.float32)
    o_ref[...] = o + bd_ref[...]


def _mlp_ep(za, zb, w1a, w1b, b1, w2, b2, w3, b3, wda, wdb, bd):
    e, dt = za.shape
    lat = w1a.shape[0]
    hid = w2.shape[0]
    de = w3.shape[1]
    row = pl.BlockSpec((TE, dt), lambda i: (i, 0))
    return pl.pallas_call(
        functools.partial(_ep_body, lat),
        grid=(e // TE,),
        in_specs=[row, row,
                  _fixed((lat, hid)), _fixed((lat, hid)), _fixed((1, hid)),
                  _fixed((hid, hid)), _fixed((1, hid)),
                  _fixed((hid, de)), _fixed((1, de)),
                  _fixed((lat, de)), _fixed((lat, de)), _fixed((1, de))],
        out_specs=pl.BlockSpec((TE, de), lambda i: (i, 0)),
        out_shape=jax.ShapeDtypeStruct((e, de), jnp.float32),
    )(za, zb, w1a, w1b, b1, w2, b2, w3, b3, wda, wdb, bd)


def _sc_mesh():
    return plsc.VectorSubcoreMesh(core_axis_name="c", subcore_axis_name="s")


def _gather_rows(table, idx):
    n, d = table.shape
    e = idx.shape[0]
    idx2 = idx.reshape(1, e)

    def outer(x_hbm, i_hbm, o_hbm):
        def body(i_vmem, o_vmem):
            pltpu.sync_copy(x_hbm.at[i_vmem.at[0]], o_vmem)

        pltpu.emit_pipeline(
            body,
            grid=(e // GW,),
            in_specs=[pl.BlockSpec((1, GW), lambda i: (0, i))],
            out_specs=[pl.BlockSpec((GW, d), lambda i: (i, 0))],
            core_axis_name=("c", "s"),
            dimension_semantics=(pltpu.PARALLEL,),
        )(i_hbm, o_hbm)

    return pl.kernel(
        outer,
        out_type=jax.ShapeDtypeStruct((e, d), table.dtype),
        mesh=_sc_mesh(),
    )(table, idx2)


def _scatter_sum(m2, dst, n):
    _, e, hh = m2.shape
    ns = 16
    ep = e // ns
    nch = ep // SCH
    nw = 10
    npw = n // nw
    zr = 200

    def body(m_hbm, d_hbm, cst_hbm, o_hbm, ia_v, ib_v, ra_v, rb_v, zb_v,
             acc_sh, sem0, sem1):
        c = lax.axis_index("c")
        s = lax.axis_index("s")

        pltpu.sync_copy(cst_hbm, zb_v)

        @pl.when(s < nw)
        def _():
            @pl.loop(0, npw, step=zr)
            def _(r):
                pltpu.sync_copy(zb_v, acc_sh.at[pl.ds(s * npw + r, zr)])

        plsc.subcore_barrier()

        base = s * ep
        sems = (sem0, sem1)
        idxs = (ia_v, ib_v)
        rows = (ra_v, rb_v)

        def start(kk, b):
            off = base + kk * SCH
            pltpu.async_copy(d_hbm.at[pl.ds(off, SCH)], idxs[b], sems[b])
            pltpu.async_copy(m_hbm.at[c, pl.ds(off, SCH)], rows[b], sems[b])

        def wait(b):
            pltpu.make_async_copy(d_hbm.at[pl.ds(0, SCH)], idxs[b],
                                  sems[b]).wait()
            pltpu.make_async_copy(m_hbm.at[0, pl.ds(0, SCH)], rows[b],
                                  sems[b]).wait()

        start(0, 0)

        @pl.loop(0, nch + 1, step=2)
        def _(k):
            for b in range(2):
                kk = k + b

                @pl.when(kk < nch)
                def _():
                    @pl.when(kk + 1 < nch)
                    def _():
                        start(kk + 1, 1 - b)

                    wait(b)
                    pltpu.sync_copy(rows[b], acc_sh.at[idxs[b]], add=True)

        plsc.subcore_barrier()

        @pl.when(s < nw)
        def _():
            @pl.loop(0, npw, step=zr)
            def _(r):
                pltpu.sync_copy(acc_sh.at[pl.ds(s * npw + r, zr)], zb_v)
                pltpu.sync_copy(zb_v, o_hbm.at[c, pl.ds(s * npw + r, zr)])

    return pl.kernel(
        body,
        out_type=jax.ShapeDtypeStruct((2, n, hh), jnp.float32),
        mesh=_sc_mesh(),
        scratch_types=[
            pltpu.VMEM((SCH,), jnp.int32),
            pltpu.VMEM((SCH,), jnp.int32),
            pltpu.VMEM((SCH, hh), jnp.float32),
            pltpu.VMEM((SCH, hh), jnp.float32),
            pltpu.VMEM((zr, hh), jnp.float32),
            pltpu.VMEM_SHARED((n, hh), jnp.float32),
            pltpu.SemaphoreType.DMA,
            pltpu.SemaphoreType.DMA,
        ],
    )(m2, dst, jnp.zeros((zr, hh), jnp.float32))


def _edge_counts16(dst, n):
    e = dst.shape[0]
    ns = 16
    ep = e // ns
    nch = ep // SCH
    nw = 10
    npw = n // nw
    zr = 200

    def body(d_hbm, cst_hbm, o_hbm, idx_v, ones_v, zb_v, acc_sh, sem):
        c = lax.axis_index("c")
        s = lax.axis_index("s")

        pltpu.sync_copy(cst_hbm.at[pl.ds(0, SCH)], ones_v)
        pltpu.sync_copy(cst_hbm.at[pl.ds(SCH, zr)], zb_v)

        @pl.when(s < nw)
        def _():
            @pl.loop(0, npw, step=zr)
            def _(r):
                pltpu.sync_copy(zb_v, acc_sh.at[pl.ds(s * npw + r, zr)])

        plsc.subcore_barrier()

        base = s * ep

        @pl.loop(0, nch)
        def _(k):
            pltpu.sync_copy(d_hbm.at[pl.ds(base + k * SCH, SCH)], idx_v)
            pltpu.sync_copy(ones_v, acc_sh.at[idx_v], add=True)

        plsc.subcore_barrier()

        @pl.when((c == 0) & (s < nw))
        def _():
            @pl.loop(0, npw, step=zr)
            def _(r):
                pltpu.sync_copy(acc_sh.at[pl.ds(s * npw + r, zr)], zb_v)
                pltpu.sync_copy(zb_v, o_hbm.at[pl.ds(s * npw + r, zr)])

    return pl.kernel(
        body,
        out_type=jax.ShapeDtypeStruct((n, 16), jnp.float32),
        mesh=_sc_mesh(),
        scratch_types=[
            pltpu.VMEM((SCH,), jnp.int32),
            pltpu.VMEM((SCH, 16), jnp.float32),
            pltpu.VMEM((zr, 16), jnp.float32),
            pltpu.VMEM_SHARED((n, 16), jnp.float32),
            pltpu.SemaphoreType.DMA,
        ],
    )(dst, jnp.concatenate([jnp.ones((SCH, 16), jnp.float32),
                            jnp.zeros((zr, 16), jnp.float32)]))


def _t(lp):
    return jnp.asarray(lp[0]).T, jnp.asarray(lp[1])[None, :]


def _jnp_scatter(m2, dst, n):
    _, e, hh = m2.shape
    msg = jnp.concatenate([m2[0], m2[1]], axis=-1)
    s = jax.ops.segment_sum(msg, dst, num_segments=n)
    return jnp.stack([s[:, :hh], s[:, hh:]])


def kernel(x, edge_index, edge_attr, params):
    n, din = x.shape
    e = edge_index.shape[1]
    src = edge_index[0]
    dst = edge_index[1]
    p = params

    cnt16 = _edge_counts16(dst, n)

    h = x
    for i in range(3):
        blk = p['eblk%d' % i]
        gamma, beta = p['ebn'][i]
        d = h.shape[1]
        st = _stats(h)
        s, sh = _bn_affine(st, gamma, beta, float(n))
        w1, b1 = _t(blk['nn1'])
        w1x, w1e = w1[:d], w1[d:]
        w2, b2 = _t(blk['nn2'])
        xi = _gather_rows(h, dst)
        m = _mlp_enc(xi, edge_attr, s, sh, w1x, w1e, b1, w2, b2, True)
        sums = _scatter_sum(m, dst, n)
        if 'res' in blk:
            wr, br = _t(blk['res'])
        else:
            wr, br = None, None
        h, _ = _combine(sums, cnt16, h, s, sh, wr, br, w2.shape[1])

    wmu, bmu = _t(p['fc_mu'])
    wlv, blv = _t(p['fc_logvar'])
    eps = jax.random.normal(jax.random.key(42), (n, wmu.shape[1]),
                            dtype=jnp.float32)
    mu, logvar, z, zp, zst = _znodes(h, eps, wmu, bmu, wlv, blv)

    h = z
    prev_st = zst
    zi = None
    zj = None
    for i in range(2):
        blk = p['dblk%d' % i]
        gamma, beta = p['dbn'][i]
        d = h.shape[1]
        final = i == 1
        s, sh = _bn_affine(prev_st, gamma, beta, float(n))
        w1, b1 = _t(blk['nn1'])
        w1a, w1b = w1[:d], w1[d:]
        w2, b2 = _t(blk['nn2'])
        gtab = zp if i == 0 else h
        xi = _gather_rows(gtab, dst)
        xj = _gather_rows(gtab, src)
        if i == 0:
            zi, zj = xi, xj
        m = _mlp_dec(xi, xj, s, sh, w1a, w1b, b1, w2, b2, not final)
        sums = _scatter_sum(m, dst, n)
        wr, br = _t(blk['res'])
        h, prev_st = _combine(sums, cnt16, h, s, sh, wr, br, w2.shape[1])
    x_recon = h

    ep = p['ep']
    w1, b1 = _t(ep['fc1'])
    lat = z.shape[1]
    w1a, w1b = w1[:lat], w1[lat:]
    w2, b2 = _t(ep['fc2'])
    w3, b3 = _t(ep['fc3'])
    wd, bd = _t(ep['fc_direct'])
    wda, wdb = wd[:lat], wd[lat:]
    pred_edge = _mlp_ep(zj, zi, w1a, w1b, b1, w2, b2, w3, b3, wda, wdb, bd)

    return x_recon, pred_edge, mu, logvar
